# Initial kernel scaffold; baseline (speedup 1.0000x reference)
import jax
import jax.numpy as jnp
from jax.experimental import pallas as pl


def kernel(nodes_l0, coords, edges, coeffs0, coeffs1, coeffs2, w0, w1, w2, senders, receivers):
    raise NotImplementedError("write your pallas kernel here")


# speedup vs baseline: 19.1515x; 19.1515x over previous
import functools

import jax
import jax.numpy as jnp
from jax import lax
from jax.experimental import pallas as pl
from jax.experimental.pallas import tpu as pltpu

_NUM_BASIS = 5
_MAX_CENTER = 3.5
_SPREAD = _MAX_CENTER / _NUM_BASIS
_CSTEP = _MAX_CENTER / (_NUM_BASIS - 1)


def _body(edges_ref, snd_ref, rcv_ref, c0_ref, c1_ref, c2_ref,
          w0_ref, w1_ref, w2_ref, nodes_ref, outa_ref, outb_ref,
          m8_ref, mb_ref, *, block_e):
    i = pl.program_id(0)

    @pl.when(i == 0)
    def _init():
        outa_ref[...] = jnp.zeros_like(outa_ref)
        outb_ref[...] = jnp.zeros_like(outb_ref)

    B = block_e
    xyz = edges_ref[...]
    x = xyz[0:1, :]
    y = xyz[1:2, :]
    z = xyz[2:3, :]
    r2 = x * x + y * y + z * z
    r = jnp.sqrt(r2)

    centers = lax.broadcasted_iota(
        jnp.int32, (_NUM_BASIS, 1), 0).astype(jnp.float32) * _CSTEP
    rbf = jnp.exp(-_SPREAD * (r - centers) ** 2)

    dn = (((0,), (0,)), ((), ()))
    f32 = jnp.float32
    r0w = lax.dot_general(rbf, c0_ref[...], dn, preferred_element_type=f32)
    r1w = lax.dot_general(rbf, c1_ref[...], dn, preferred_element_type=f32)
    r2w = lax.dot_general(rbf, c2_ref[...], dn, preferred_element_type=f32)
    r0w = r0w * w0_ref[...]
    r1w = r1w * w1_ref[...]
    r2w = r2w * w2_ref[...]

    s3 = 1.7320508075688772
    y2 = jnp.concatenate([
        s3 * x * y,
        s3 * y * z,
        0.5 * (3.0 * z * z - r2),
        s3 * x * z,
        0.5 * s3 * (x * x - y * y),
    ], axis=0)

    y1t = xyz.T
    y2t = y2.T

    m0 = r0w[:, None, :]
    m1 = r1w[:, None, :] * y1t[:, :, None]
    m2a = r2w[:, None, :] * y2t[:, 0:4, None]
    m8 = jnp.concatenate([m0, m1, m2a], axis=1)
    m8_ref[...] = m8.reshape(B * 8, 128)
    mb_ref[...] = r2w * y2t[:, 4:5]

    def step(e, _):
        s = snd_ref[0, 0, e]
        n = rcv_ref[0, 0, e]
        g = nodes_ref[pl.ds(s, 1), :]
        outa_ref[pl.ds(n * 8, 8), :] += g * m8_ref[pl.ds(e * 8, 8), :]
        outb_ref[pl.ds(n, 1), :] += g * mb_ref[pl.ds(e, 1), :]
        return 0

    lax.fori_loop(0, B, step, 0)


def kernel(nodes_l0, coords, edges, coeffs0, coeffs1, coeffs2,
           w0, w1, w2, senders, receivers):
    n_nodes, n_c = nodes_l0.shape[0], nodes_l0.shape[1]
    n_edges = senders.shape[0]
    B = 512
    nb = n_edges // B

    nodes2d = nodes_l0[:, :, 0]
    edges_t = edges.T
    snd = senders.reshape(nb, 1, B)
    rcv = receivers.reshape(nb, 1, B)
    c0t = coeffs0.T
    c1t = coeffs1.T
    c2t = coeffs2.T
    w0r = w0.reshape(1, n_c)
    w1r = w1.reshape(1, n_c)
    w2r = w2.reshape(1, n_c)

    full = lambda shape: pl.BlockSpec(shape, lambda i: (0,) * len(shape))
    buf = pl.pallas_call(
        functools.partial(_body, block_e=B),
        grid=(nb,),
        in_specs=[
            pl.BlockSpec((3, B), lambda i: (0, i)),
            pl.BlockSpec((1, 1, B), lambda i: (i, 0, 0),
                         memory_space=pltpu.SMEM),
            pl.BlockSpec((1, 1, B), lambda i: (i, 0, 0),
                         memory_space=pltpu.SMEM),
            full((_NUM_BASIS, n_c)),
            full((_NUM_BASIS, n_c)),
            full((_NUM_BASIS, n_c)),
            full((1, n_c)),
            full((1, n_c)),
            full((1, n_c)),
            full((n_nodes, n_c)),
        ],
        out_specs=[full((8 * n_nodes, n_c)), full((n_nodes, n_c))],
        out_shape=[
            jax.ShapeDtypeStruct((8 * n_nodes, n_c), jnp.float32),
            jax.ShapeDtypeStruct((n_nodes, n_c), jnp.float32),
        ],
        scratch_shapes=[
            pltpu.VMEM((B * 8, n_c), jnp.float32),
            pltpu.VMEM((B, n_c), jnp.float32),
        ],
        compiler_params=pltpu.CompilerParams(
            dimension_semantics=("arbitrary",),
        ),
    )(edges_t, snd, rcv, c0t, c1t, c2t, w0r, w1r, w2r, nodes2d)

    bufa, bufb = buf
    bufa = bufa.reshape(n_nodes, 8, n_c)
    out0 = jnp.transpose(bufa[:, 0:1, :], (0, 2, 1))
    out1 = jnp.transpose(bufa[:, 1:4, :], (0, 2, 1))
    out2 = jnp.transpose(
        jnp.concatenate([bufa[:, 4:8, :], bufb[:, None, :]], axis=1),
        (0, 2, 1))
    return (coords, out0, out1, out2)


# speedup vs baseline: 27.2002x; 1.4203x over previous
import functools

import jax
import jax.numpy as jnp
from jax import lax
from jax.experimental import pallas as pl
from jax.experimental.pallas import tpu as pltpu

_NUM_BASIS = 5
_MAX_CENTER = 3.5
_SPREAD = _MAX_CENTER / _NUM_BASIS
_CSTEP = _MAX_CENTER / (_NUM_BASIS - 1)


def _body(edges_ref, snd_ref, rcv_ref, c0_ref, c1_ref, c2_ref,
          w0_ref, w1_ref, w2_ref, nodes_ref, outa_ref, outb_ref,
          m8_ref, mb_ref, *, block_e):
    i = pl.program_id(0)

    @pl.when(i == 0)
    def _init():
        outa_ref[...] = jnp.zeros_like(outa_ref)
        outb_ref[...] = jnp.zeros_like(outb_ref)

    B = block_e
    xyz = edges_ref[...]
    x = xyz[0:1, :]
    y = xyz[1:2, :]
    z = xyz[2:3, :]
    r2 = x * x + y * y + z * z
    r = jnp.sqrt(r2)

    centers = lax.broadcasted_iota(
        jnp.int32, (_NUM_BASIS, 1), 0).astype(jnp.float32) * _CSTEP
    rbf = jnp.exp(-_SPREAD * (r - centers) ** 2)

    dn = (((0,), (0,)), ((), ()))
    f32 = jnp.float32
    r0w = lax.dot_general(rbf, c0_ref[...], dn, preferred_element_type=f32)
    r1w = lax.dot_general(rbf, c1_ref[...], dn, preferred_element_type=f32)
    r2w = lax.dot_general(rbf, c2_ref[...], dn, preferred_element_type=f32)
    r0w = r0w * w0_ref[...]
    r1w = r1w * w1_ref[...]
    r2w = r2w * w2_ref[...]

    s3 = 1.7320508075688772
    y2 = jnp.concatenate([
        s3 * x * y,
        s3 * y * z,
        0.5 * (3.0 * z * z - r2),
        s3 * x * z,
        0.5 * s3 * (x * x - y * y),
    ], axis=0)

    y1t = xyz.T
    y2t = y2.T

    m0 = r0w[:, None, :]
    m1 = r1w[:, None, :] * y1t[:, :, None]
    m2a = r2w[:, None, :] * y2t[:, 0:4, None]
    m8 = jnp.concatenate([m0, m1, m2a], axis=1)
    m8_ref[...] = m8.reshape(B * 8, 128)
    mb_ref[...] = r2w * y2t[:, 4:5]

    def step(e, _):
        s = snd_ref[0, 0, e]
        n = rcv_ref[0, 0, e]
        g = nodes_ref[pl.ds(s, 1), :]
        outa_ref[pl.ds(n * 8, 8), :] += g * m8_ref[pl.ds(e * 8, 8), :]
        outb_ref[pl.ds(n, 1), :] += g * mb_ref[pl.ds(e, 1), :]
        return 0

    lax.fori_loop(0, B, step, 0, unroll=4)


def kernel(nodes_l0, coords, edges, coeffs0, coeffs1, coeffs2,
           w0, w1, w2, senders, receivers):
    n_nodes, n_c = nodes_l0.shape[0], nodes_l0.shape[1]
    n_edges = senders.shape[0]
    B = 512
    nb = n_edges // B

    nodes2d = nodes_l0[:, :, 0]
    edges_t = edges.T
    snd = senders.reshape(nb, 1, B)
    rcv = receivers.reshape(nb, 1, B)
    c0t = coeffs0.T
    c1t = coeffs1.T
    c2t = coeffs2.T
    w0r = w0.reshape(1, n_c)
    w1r = w1.reshape(1, n_c)
    w2r = w2.reshape(1, n_c)

    full = lambda shape: pl.BlockSpec(shape, lambda i: (0,) * len(shape))
    buf = pl.pallas_call(
        functools.partial(_body, block_e=B),
        grid=(nb,),
        in_specs=[
            pl.BlockSpec((3, B), lambda i: (0, i)),
            pl.BlockSpec((1, 1, B), lambda i: (i, 0, 0),
                         memory_space=pltpu.SMEM),
            pl.BlockSpec((1, 1, B), lambda i: (i, 0, 0),
                         memory_space=pltpu.SMEM),
            full((_NUM_BASIS, n_c)),
            full((_NUM_BASIS, n_c)),
            full((_NUM_BASIS, n_c)),
            full((1, n_c)),
            full((1, n_c)),
            full((1, n_c)),
            full((n_nodes, n_c)),
        ],
        out_specs=[full((8 * n_nodes, n_c)), full((n_nodes, n_c))],
        out_shape=[
            jax.ShapeDtypeStruct((8 * n_nodes, n_c), jnp.float32),
            jax.ShapeDtypeStruct((n_nodes, n_c), jnp.float32),
        ],
        scratch_shapes=[
            pltpu.VMEM((B * 8, n_c), jnp.float32),
            pltpu.VMEM((B, n_c), jnp.float32),
        ],
        compiler_params=pltpu.CompilerParams(
            dimension_semantics=("arbitrary",),
        ),
    )(edges_t, snd, rcv, c0t, c1t, c2t, w0r, w1r, w2r, nodes2d)

    bufa, bufb = buf
    bufa = bufa.reshape(n_nodes, 8, n_c)
    out0 = jnp.transpose(bufa[:, 0:1, :], (0, 2, 1))
    out1 = jnp.transpose(bufa[:, 1:4, :], (0, 2, 1))
    out2 = jnp.transpose(
        jnp.concatenate([bufa[:, 4:8, :], bufb[:, None, :]], axis=1),
        (0, 2, 1))
    return (coords, out0, out1, out2)
